# baseline (speedup 1.0000x reference)
import functools

import jax
import jax.numpy as jnp
from jax import lax
from jax.experimental import pallas as pl
from jax.experimental.pallas import tpu as pltpu
from jax.experimental.pallas import tpu_sc as plsc

NC = 2
NS = 16
NW = NC * NS
CHUNK = 128


def _emb_body(n_chunks, table_hbm, idx_hbm, out_hbm, idx_v, rows_v,
              gsem0, gsem1):
    wid = lax.axis_index("s") * NC + lax.axis_index("c")
    chunk_base = wid * n_chunks

    pltpu.sync_copy(idx_hbm.at[wid], idx_v)

    gsems = (gsem0, gsem1)

    def gather_start(c, b):
        pltpu.async_copy(table_hbm.at[idx_v.at[c]], rows_v.at[b], gsems[b])

    def gather_wait(b):
        pltpu.make_async_copy(table_hbm.at[idx_v.at[0]], rows_v.at[b],
                              gsems[b]).wait()

    def write_out(c, b):
        row_off = (chunk_base + c) * CHUNK
        pltpu.sync_copy(rows_v.at[b], out_hbm.at[pl.ds(row_off, CHUNK)])

    gather_start(0, 0)
    gather_start(1, 1)

    def body(i, carry):
        for b in range(2):
            c = 2 * i + b
            gather_wait(b)
            write_out(c, b)
            gather_start(c + 2, b)
        return carry

    lax.fori_loop(0, n_chunks // 2 - 1, body, 0, unroll=False)

    for b in range(2):
        c = n_chunks - 2 + b
        gather_wait(b)
        write_out(c, b)


def kernel(token_ids, weights):
    rows, cols = token_ids.shape
    b_total = rows * cols
    d = weights.shape[1]
    assert b_total % (NW * CHUNK) == 0
    n_chunks = b_total // (NW * CHUNK)

    idx3 = token_ids.reshape(NW, n_chunks, CHUNK).astype(jnp.int32)

    emb = functools.partial(
        pl.kernel,
        mesh=plsc.VectorSubcoreMesh(core_axis_name="c", subcore_axis_name="s"),
        out_type=jax.ShapeDtypeStruct((b_total, d), jnp.float32),
        scratch_types=[
            pltpu.VMEM((n_chunks, CHUNK), jnp.int32),
            pltpu.VMEM((2, CHUNK, d), jnp.float32),
            pltpu.SemaphoreType.DMA,
            pltpu.SemaphoreType.DMA,
        ],
        compiler_params=pltpu.CompilerParams(use_tc_tiling_on_sc=False),
    )(functools.partial(_emb_body, n_chunks))

    out = emb(weights, idx3)
    return out.reshape(rows, cols, d)


# speedup vs baseline: 1.0709x; 1.0709x over previous
import functools

import jax
import jax.numpy as jnp
from jax import lax
from jax.experimental import pallas as pl
from jax.experimental.pallas import tpu as pltpu
from jax.experimental.pallas import tpu_sc as plsc

NC = 2
NS = 16
NW = NC * NS
BB = 128
CH = 32
L = 16


def _emb_body(seq, d, table_hbm, idx_hbm, out_hbm, idx_v, rows_v, obuf,
              gsem0, gsem1, wsem0, wsem1):
    wid = lax.axis_index("s") * NC + lax.axis_index("c")

    pltpu.sync_copy(idx_hbm.at[wid], idx_v)

    gsems = (gsem0, gsem1)
    wsems = (wsem0, wsem1)

    def issue(s, h, slot):
        def g16(g, carry):
            vec = idx_v[s, 0, pl.ds(h * CH + g * L, L)]
            for u in range(L):
                i = vec[u]
                tb = pl.multiple_of((i >> 3) * 8, 8)
                j = g * L + u
                pltpu.async_copy(
                    table_hbm.at[pl.ds(tb, 8), :],
                    rows_v.at[slot, pl.ds(j * 8, 8), :],
                    gsems[slot],
                )
            return carry

        lax.fori_loop(0, CH // L, g16, 0, unroll=True)

    def drain(slot):
        def one(j, carry):
            pltpu.make_async_copy(table_hbm.at[pl.ds(0, 8), :],
                                  rows_v.at[slot, pl.ds(0, 8), :],
                                  gsems[slot]).wait()
            return carry

        lax.fori_loop(0, CH, one, 0, unroll=False)

    def transpose(s, h, slot, obs):
        rows = rows_v.at[slot]
        for g in range(CH // L):
            vec = idx_v[s, 0, pl.ds(h * CH + g * L, L)]
            rowv = (lax.iota(jnp.int32, L) + g * L) * 8 + (vec & 7)

            def per_f(f, carry):
                colv = jnp.zeros((L,), jnp.int32) + f
                vals = plsc.load_gather(rows, [rowv, colv])
                obuf[obs, f, pl.ds(h * CH + g * L, L)] = vals
                return carry

            lax.fori_loop(0, d, per_f, 0, unroll=False)

    def write(s, obs):
        pltpu.async_copy(obuf.at[obs], out_hbm.at[s, :, pl.ds(wid * BB, BB)],
                         wsems[obs])

    def wwait(obs):
        pltpu.make_async_copy(obuf.at[obs], out_hbm.at[0, :, pl.ds(0, BB)],
                              wsems[obs]).wait()

    issue(0, 0, 0)
    issue(0, 1, 1)

    def body(i, carry):
        for obs in range(2):
            s = 2 * i + obs

            @pl.when(s >= 2)
            def _():
                wwait(obs)

            for h in range(4):
                slot = h % 2
                drain(slot)
                transpose(s, h, slot, obs)
                if h < 2:
                    issue(s, h + 2, slot)
                else:
                    @pl.when(s + 1 < seq)
                    def _():
                        issue(s + 1, h - 2, slot)

            write(s, obs)
        return carry

    lax.fori_loop(0, seq // 2, body, 0, unroll=False)

    wwait(0)
    wwait(1)


def kernel(token_ids, weights):
    bsz, seq = token_ids.shape
    n, d = weights.shape
    assert bsz == NW * BB and d == 64 and seq % 2 == 0

    idx4 = (token_ids.astype(jnp.int32)
            .reshape(NW, BB, seq)
            .transpose(0, 2, 1)
            .reshape(NW, seq, 1, BB))

    emb = functools.partial(
        pl.kernel,
        mesh=plsc.VectorSubcoreMesh(core_axis_name="c", subcore_axis_name="s"),
        out_type=jax.ShapeDtypeStruct((seq, d, bsz), jnp.float32),
        scratch_types=[
            pltpu.VMEM((seq, 1, BB), jnp.int32),
            pltpu.VMEM((2, CH * 8, d), jnp.float32),
            pltpu.VMEM((2, d, BB), jnp.float32),
            pltpu.SemaphoreType.DMA,
            pltpu.SemaphoreType.DMA,
            pltpu.SemaphoreType.DMA,
            pltpu.SemaphoreType.DMA,
        ],
        compiler_params=pltpu.CompilerParams(needs_layout_passes=False),
    )(functools.partial(_emb_body, seq, d))

    out_t = emb(weights, idx4)
    return out_t.transpose(2, 0, 1)
